# baseline (speedup 1.0000x reference)
import functools

import jax
import jax.numpy as jnp
from jax import lax
from jax.experimental import pallas as pl
from jax.experimental.pallas import tpu as pltpu
from jax.experimental.pallas import tpu_sc as plsc

_NUM_CORES = 2
_NUM_SUBCORES = 16
_NUM_WORKERS = _NUM_CORES * _NUM_SUBCORES


@functools.partial(jax.jit, static_argnums=(1, 2))
def _position_embedding(table, seq_len, dim):
    rows_per_worker = seq_len // _NUM_WORKERS
    mesh = plsc.VectorSubcoreMesh(
        core_axis_name="c", subcore_axis_name="s", num_cores=_NUM_CORES
    )

    @functools.partial(
        pl.kernel,
        out_type=jax.ShapeDtypeStruct((seq_len, dim), table.dtype),
        mesh=mesh,
    )
    def copy_kernel(table_hbm, out_hbm):
        wid = lax.axis_index("s") * _NUM_CORES + lax.axis_index("c")
        base = wid * rows_per_worker
        pltpu.sync_copy(
            table_hbm.at[pl.ds(base, rows_per_worker)],
            out_hbm.at[pl.ds(base, rows_per_worker)],
        )

    return copy_kernel(table)


def kernel(inputs, table):
    seq_len = inputs.shape[-1]
    return _position_embedding(table, seq_len, table.shape[1])


# speedup vs baseline: 16.7184x; 16.7184x over previous
import functools

import jax
import jax.numpy as jnp
from jax import lax
from jax.experimental import pallas as pl
from jax.experimental.pallas import tpu as pltpu
from jax.experimental.pallas import tpu_sc as plsc

_NUM_CORES = 2
_NUM_SUBCORES = 16
_NUM_WORKERS = _NUM_CORES * _NUM_SUBCORES
_CHUNK_ROWS = 32


@functools.partial(jax.jit, static_argnums=(1, 2))
def _position_embedding(table, seq_len, dim):
    rows_per_worker = seq_len // _NUM_WORKERS
    n_chunks = rows_per_worker // _CHUNK_ROWS
    mesh = plsc.VectorSubcoreMesh(
        core_axis_name="c", subcore_axis_name="s", num_cores=_NUM_CORES
    )

    @functools.partial(
        pl.kernel,
        out_type=jax.ShapeDtypeStruct((seq_len, dim), table.dtype),
        mesh=mesh,
        scratch_types=[
            pltpu.VMEM((2, _CHUNK_ROWS, dim), table.dtype),
            pltpu.SemaphoreType.DMA((2,)),
            pltpu.SemaphoreType.DMA((2,)),
        ],
    )
    def copy_kernel(table_hbm, out_hbm, buf, in_sems, out_sems):
        wid = lax.axis_index("s") * _NUM_CORES + lax.axis_index("c")
        base = wid * rows_per_worker

        def chunk_in(c):
            return table_hbm.at[pl.ds(base + c * _CHUNK_ROWS, _CHUNK_ROWS)]

        def chunk_out(c):
            return out_hbm.at[pl.ds(base + c * _CHUNK_ROWS, _CHUNK_ROWS)]

        in_dma = {}
        out_dma = {}
        in_dma[0] = pltpu.async_copy(chunk_in(0), buf.at[0], in_sems.at[0])
        for c in range(n_chunks):
            b = c % 2
            nb = (c + 1) % 2
            if c + 1 < n_chunks:
                if c - 1 >= 0:
                    out_dma[c - 1].wait()
                in_dma[c + 1] = pltpu.async_copy(
                    chunk_in(c + 1), buf.at[nb], in_sems.at[nb]
                )
            in_dma[c].wait()
            out_dma[c] = pltpu.async_copy(buf.at[b], chunk_out(c), out_sems.at[b])
        if n_chunks >= 2:
            out_dma[n_chunks - 2].wait()
        out_dma[n_chunks - 1].wait()

    return copy_kernel(table)


def kernel(inputs, table):
    seq_len = inputs.shape[-1]
    return _position_embedding(table, seq_len, table.shape[1])
